# Initial kernel scaffold; baseline (speedup 1.0000x reference)
import jax
import jax.numpy as jnp
from jax.experimental import pallas as pl


def kernel(coordinates, nuclear_charges, natom_counts):
    raise NotImplementedError("write your pallas kernel here")


# speedup vs baseline: 4.5752x; 4.5752x over previous
import functools
import math

import jax
import jax.numpy as jnp
from jax import lax
from jax.experimental import pallas as pl
from jax.experimental.pallas import tpu as pltpu
from jax.experimental.pallas import tpu_sc as plsc

_B, _N = 16, 64
_NPAD = 640
_LIDX = (0, 1, 1, 1, 2, 2, 2, 2, 2, 2)
_AW = (1.0, 1.0, 1.0, 1.0, 1.0, 2.0, 1.0, 2.0, 2.0, 1.0)
_PAIRS = ((0, 1), (0, 2), (0, 3), (1, 2), (1, 3), (2, 3))
_KRAD = 0.7978845608028654


_GATHER_DNUMS = lax.GatherDimensionNumbers(
    offset_dims=(), collapsed_slice_dims=(0,), start_index_map=(0,))


def _take(v, idx):
    return lax.gather(
        v, idx[:, None], _GATHER_DNUMS, (1,),
        mode=lax.GatherScatterMode.PROMISE_IN_BOUNDS)


def _fp_body(coords_hbm, chg_hbm, nat_hbm, out_hbm, cflat_v, chg_v, nat_v,
             x_v, y_v, z_v, ang_v, dist_v, kcw_v, t_v, stage_v):
    cid = lax.axis_index("c")
    sid = lax.axis_index("s")
    wid = sid * 2 + cid
    b = wid // 2
    i0 = (wid % 2) * 32

    pltpu.sync_copy(coords_hbm.at[b], cflat_v)
    pltpu.sync_copy(chg_hbm.at[b], chg_v)
    pltpu.sync_copy(nat_hbm, nat_v)

    lane_i = lax.iota(jnp.int32, 16)
    lane_f = lane_i.astype(jnp.float32)
    pwf = jnp.left_shift(1, lane_i).astype(jnp.float32)
    off0 = (lane_f + 1.0) * 0.3
    off1 = (lane_f + 17.0) * 0.3
    zero16 = jnp.zeros((16,), jnp.float32)

    for c4 in range(4):
        src = 3 * (c4 * 16 + lane_i)
        x_v[pl.ds(c4 * 16, 16)] = plsc.load_gather(cflat_v, [src])
        y_v[pl.ds(c4 * 16, 16)] = plsc.load_gather(cflat_v, [src + 1])
        z_v[pl.ds(c4 * 16, 16)] = plsc.load_gather(cflat_v, [src + 2])

    natoms_v = _take(nat_v[pl.ds(0, 16)], jnp.broadcast_to(b, (16,)))

    def atom_body(il, carry):
        ig = i0 + il
        igv = jnp.broadcast_to(ig, (16,))
        xi = _take(x_v[pl.ds((ig // 16) * 16, 16)], jnp.broadcast_to(ig % 16, (16,)))
        yi = _take(y_v[pl.ds((ig // 16) * 16, 16)], jnp.broadcast_to(ig % 16, (16,)))
        zi = _take(z_v[pl.ds((ig // 16) * 16, 16)], jnp.broadcast_to(ig % 16, (16,)))

        for r in range(80):
            t_v[pl.ds(16 * r, 16)] = zero16

        okbits = []
        slobits = []
        shibits = []
        for c4 in range(4):
            sl = pl.ds(c4 * 16, 16)
            jv = lane_i + (c4 * 16)
            q = chg_v[sl]
            dx = xi - x_v[sl]
            dy = yi - y_v[sl]
            dz = zi - z_v[sl]
            d2 = dx * dx + dy * dy + dz * dz
            s = jnp.where(q == 6, 1, jnp.where(q == 7, 2,
                                               jnp.where(q == 8, 3, 0)))
            is_sp = (q == 1) | (q == 6) | (q == 7) | (q == 8)
            ok = (jv != igv) & (jv < natoms_v) & (d2 < 36.0) & is_sp
            okbits.append(jnp.sum(jnp.where(ok, pwf, 0.0)).astype(jnp.int32))
            slobits.append(jnp.sum(
                jnp.where(ok & ((s & 1) == 1), pwf, 0.0)).astype(jnp.int32))
            shibits.append(jnp.sum(
                jnp.where(ok & (s >= 2), pwf, 0.0)).astype(jnp.int32))

            d2s = jnp.where(ok, d2, 1.0)
            bits = lax.bitcast_convert_type(d2s, jnp.int32)
            yv = lax.bitcast_convert_type(0x5F3759DF - (bits >> 1), jnp.float32)
            for _ in range(3):
                yv = yv * (1.5 - 0.5 * d2s * yv * yv)
            dist = d2s * yv
            rinv2 = 1.0 / d2s
            rinv3 = rinv2 * yv
            rinv4 = rinv2 * rinv2
            ya = dist * (math.pi / 12.0)
            a2 = ya * ya
            cp = 1.0 + a2 * (-0.5 + a2 * (1.0 / 24.0 + a2 * (
                -1.0 / 720.0 + a2 * (1.0 / 40320.0 + a2 * (-1.0 / 3628800.0)))))
            kcw = jnp.where(ok, _KRAD * (cp * cp), 0.0)
            dist_v[sl] = dist
            kcw_v[sl] = kcw
            ang_v[0, sl] = rinv2
            ang_v[1, sl] = rinv3 * dx
            ang_v[2, sl] = rinv3 * dy
            ang_v[3, sl] = rinv3 * dz
            ang_v[4, sl] = rinv4 * dx * dx
            ang_v[5, sl] = rinv4 * dx * dy
            ang_v[6, sl] = rinv4 * dy * dy
            ang_v[7, sl] = rinv4 * dx * dz
            ang_v[8, sl] = rinv4 * dy * dz
            ang_v[9, sl] = rinv4 * dz * dz

        for c4 in range(4):
            sl = pl.ds(c4 * 16, 16)
            dch = dist_v[sl]
            kch = kcw_v[sl]
            ach = [ang_v[c, sl] for c in range(10)]
            okb = okbits[c4]
            slo = slobits[c4]
            shi = shibits[c4]

            def nbr(jl, carry2, dch=dch, kch=kch, ach=ach, okb=okb,
                    slo=slo, shi=shi):
                @pl.when(((okb >> jl) & 1) == 1)
                def _():
                    idx = jnp.broadcast_to(jl, (16,))
                    dj = _take(dch, idx)
                    kj = _take(kch, idx)
                    g0 = dj - off0
                    g1 = dj - off1
                    r0 = kj * jnp.exp(-2.0 * g0 * g0)
                    r1 = kj * jnp.exp(-2.0 * g1 * g1)
                    sj = ((slo >> jl) & 1) + 2 * ((shi >> jl) & 1)
                    tb = sj * 320
                    for cc in range(10):
                        a = _take(ach[cc], idx)
                        o = tb + 32 * cc
                        t_v[pl.ds(o, 16)] = t_v[pl.ds(o, 16)] + a * r0
                        t_v[pl.ds(o + 16, 16)] = t_v[pl.ds(o + 16, 16)] + a * r1
                return carry2

            lax.fori_loop(0, 16, nbr, 0)

        wiv = jnp.where(igv < natoms_v, 1.0, 0.0).astype(jnp.float32)
        for l in range(3):
            ccs = [c for c in range(10) if _LIDX[c] == l]
            for m in range(10):
                o = l * 200 + m * 20
                v0 = zero16
                v1 = zero16
                if m < 4:
                    for cc in ccs:
                        ta0 = t_v[pl.ds(m * 320 + 32 * cc, 16)]
                        ta1 = t_v[pl.ds(m * 320 + 32 * cc + 16, 16)]
                        v0 = v0 + _AW[cc] * (ta0 * ta0)
                        v1 = v1 + _AW[cc] * (ta1 * ta1)
                else:
                    sa, sb = _PAIRS[m - 4]
                    for cc in ccs:
                        w2 = 2.0 * _AW[cc]
                        v0 = v0 + w2 * (t_v[pl.ds(sa * 320 + 32 * cc, 16)]
                                        * t_v[pl.ds(sb * 320 + 32 * cc, 16)])
                        v1 = v1 + w2 * (t_v[pl.ds(sa * 320 + 32 * cc + 16, 16)]
                                        * t_v[pl.ds(sb * 320 + 32 * cc + 16, 16)])
                stage_v[pl.ds(o, 16)] = v0 * wiv
                stage_v[pl.ds(o + 16, 16)] = v1 * wiv
        pltpu.sync_copy(stage_v, out_hbm.at[b, ig])
        return carry

    lax.fori_loop(0, 32, atom_body, 0)


_fp_kernel = functools.partial(
    pl.kernel,
    out_type=jax.ShapeDtypeStruct((_B, _N, _NPAD), jnp.float32),
    mesh=plsc.VectorSubcoreMesh(core_axis_name="c", subcore_axis_name="s"),
    compiler_params=pltpu.CompilerParams(needs_layout_passes=False),
    scratch_types=[
        pltpu.VMEM((_N * 3,), jnp.float32),
        pltpu.VMEM((_N,), jnp.int32),
        pltpu.VMEM((_B,), jnp.int32),
        pltpu.VMEM((_N,), jnp.float32),
        pltpu.VMEM((_N,), jnp.float32),
        pltpu.VMEM((_N,), jnp.float32),
        pltpu.VMEM((10, _N), jnp.float32),
        pltpu.VMEM((_N,), jnp.float32),
        pltpu.VMEM((_N,), jnp.float32),
        pltpu.VMEM((1280,), jnp.float32),
        pltpu.VMEM((_NPAD,), jnp.float32),
    ],
)(_fp_body)


def kernel(coordinates, nuclear_charges, natom_counts):
    out = _fp_kernel(
        coordinates.reshape(_B, _N * 3).astype(jnp.float32),
        nuclear_charges.astype(jnp.int32),
        natom_counts.astype(jnp.int32))
    return out[:, :, :600]
